# baseline (device time: 43753 ns/iter reference)
import jax
import jax.numpy as jnp
from jax import lax
from jax.experimental import pallas as pl
from jax.experimental.pallas import tpu as pltpu

N_DEV = 8

PIPES = (
    ((1, 2, 4), 0, 96),
    ((2, 4, 1), 96, 80),
    ((4, 1, 2), 176, 80),
)
RS_BASE = (0, 4, 6)
AG_BASE = (0, 1, 3)


def _span(masks):
    out = [0]
    for m in masks:
        out = out + [e ^ m for e in out]
    return out


def kernel(t, W):
    m, k = t.shape
    _, n = W.shape
    m_per = m // N_DEV

    def body(
        t_hbm_ref,
        w_hbm_ref,
        out_ref,
        t_ref,
        acc_ref,
        w_f32_ref,
        w_bf_ref,
        recv_ref,
        rs_send_sems,
        rs_recv_sems,
        ag_send_sems,
        ag_recv_sems,
        w_dma_sem,
        t_dma_sems,
    ):
        i = lax.axis_index("i")
        l = i ^ ((i >> 1) & 1)

        def partner(mask):
            lp = l ^ mask
            return lp ^ ((lp >> 1) & 1)

        T_ORDER = (3, 7, 5, 1, 6, 2, 4, 0)

        def t_dma(o):
            rows = pl.ds((l ^ o) * m_per, m_per)
            return pltpu.make_async_copy(
                t_hbm_ref.at[rows, :], t_ref.at[rows, :],
                t_dma_sems.at[T_ORDER.index(o)],
            )

        for o in T_ORDER:
            t_dma(o).start()

        w_dma = pltpu.make_async_copy(w_hbm_ref, w_f32_ref, w_dma_sem)
        w_dma.start()

        _t_waited = set()

        def t_wait(o):
            if o not in _t_waited:
                t_dma(o).wait()
                _t_waited.add(o)

        barrier_sem = pltpu.get_barrier_semaphore()
        for mask in (1, 2, 4):
            pl.semaphore_signal(
                barrier_sem,
                inc=1,
                device_id=(partner(mask),),
                device_id_type=pl.DeviceIdType.MESH,
            )
        pl.semaphore_wait(barrier_sem, 3)

        def rs_slot(p, s, e):
            return RS_BASE[s] + _span(PIPES[p][0][s + 1 :]).index(e)

        def rs_descriptor(p, s, e):
            order, roff, h = PIPES[p]
            mask = order[s]
            slot = rs_slot(p, s, e)
            c = l ^ mask ^ e
            return pltpu.make_async_remote_copy(
                src_ref=acc_ref.at[pl.ds(c * m_per + roff, h), :],
                dst_ref=recv_ref.at[slot, pl.ds(roff, h), :],
                send_sem=rs_send_sems.at[p, slot],
                recv_sem=rs_recv_sems.at[p, slot],
                device_id=(partner(mask),),
                device_id_type=pl.DeviceIdType.MESH,
            )

        def rs_recv(p, s, e):
            rs_descriptor(p, s, e).wait()
            roff, h = PIPES[p][1], PIPES[p][2]
            return recv_ref[rs_slot(p, s, e), pl.ds(roff, h), :].astype(
                jnp.float32
            )

        def band(c, p):
            roff, h = PIPES[p][1], PIPES[p][2]
            return pl.ds(c * m_per + roff, h)

        def acc_f32(c, p):
            return acc_ref[band(c, p), :].astype(jnp.float32)


        def ag_descriptor(p, s, e, recv):
            order, roff, h = PIPES[p]
            rev = order[::-1]
            mask = rev[s]
            slot = AG_BASE[s] + _span(rev[:s]).index(e)
            c = (l ^ mask ^ e) if recv else (l ^ e)
            rows = pl.ds(c * m_per + roff, h)
            return pltpu.make_async_remote_copy(
                src_ref=out_ref.at[rows, :],
                dst_ref=out_ref.at[rows, :],
                send_sem=ag_send_sems.at[p, slot],
                recv_sem=ag_recv_sems.at[p, slot],
                device_id=(partner(mask),),
                device_id_type=pl.DeviceIdType.MESH,
            )

        def ag_recv(p, s, e):
            ag_descriptor(p, s, e, recv=True).wait()

        for p in range(3):
            ms0, ms1, ms2 = PIPES[p][0]
            for e in (ms1, ms1 ^ ms2, ms2, 0):
                t_wait(ms0 ^ e)
                c = l ^ ms0 ^ e
                acc_ref[band(c, p), :] = t_ref[band(c, p), :].astype(
                    jnp.bfloat16
                )
                rs_descriptor(p, 0, e).start()

        for p in range(3):
            ms1, ms2 = PIPES[p][0][1], PIPES[p][0][2]
            for e in (ms1, ms1 ^ ms2):
                acc_ref[band(l ^ e, p), :] = (
                    t_ref[band(l ^ e, p), :] + rs_recv(p, 0, e)
                ).astype(jnp.bfloat16)
            for e in (ms2, 0):
                rs_descriptor(p, 1, e).start()

        for p in range(3):
            ms2 = PIPES[p][0][2]
            acc_ref[band(l ^ ms2, p), :] = (
                t_ref[band(l ^ ms2, p), :]
                + rs_recv(p, 0, ms2)
                + rs_recv(p, 1, ms2)
            ).astype(jnp.bfloat16)
            rs_descriptor(p, 2, 0).start()

        w_dma.wait()
        w_bf_ref[...] = w_f32_ref[...].astype(jnp.bfloat16)

        t_wait(0)
        for p in range(3):
            reduced = (
                t_ref[band(l, p), :]
                + rs_recv(p, 0, 0)
                + rs_recv(p, 1, 0)
                + rs_recv(p, 2, 0)
            ).astype(jnp.bfloat16)
            res = jnp.dot(
                reduced, w_bf_ref[...], preferred_element_type=jnp.float32
            )
            out_ref[band(l, p), :] = res.astype(jnp.bfloat16)
            for s in (0, 1, 2):
                ag_descriptor(p, s, 0, recv=False).start()

        for p in range(3):
            rev = PIPES[p][0][::-1]
            ag_recv(p, 0, 0)
            ag_descriptor(p, 1, rev[0], recv=False).start()
            ag_descriptor(p, 2, rev[0], recv=False).start()
        for p in range(3):
            rev = PIPES[p][0][::-1]
            ag_recv(p, 1, 0)
            ag_descriptor(p, 2, rev[1], recv=False).start()
            ag_recv(p, 1, rev[0])
            ag_descriptor(p, 2, rev[1] ^ rev[0], recv=False).start()
        for p in range(3):
            rev = PIPES[p][0][::-1]
            for e in _span(rev[:2]):
                ag_recv(p, 2, e)

    return pl.pallas_call(
        body,
        out_shape=jax.ShapeDtypeStruct((m, n), jnp.bfloat16),
        in_specs=[
            pl.BlockSpec(memory_space=pl.ANY),
            pl.BlockSpec(memory_space=pl.ANY),
        ],
        out_specs=pl.BlockSpec(memory_space=pltpu.VMEM),
        scratch_shapes=[
            pltpu.VMEM((m, k), jnp.float32),
            pltpu.VMEM((m, k), jnp.bfloat16),
            pltpu.VMEM((k, n), jnp.float32),
            pltpu.VMEM((k, n), jnp.bfloat16),
            pltpu.VMEM((7, m_per, k), jnp.bfloat16),
            pltpu.SemaphoreType.DMA((3, 7)),
            pltpu.SemaphoreType.DMA((3, 7)),
            pltpu.SemaphoreType.DMA((3, 7)),
            pltpu.SemaphoreType.DMA((3, 7)),
            pltpu.SemaphoreType.DMA,
            pltpu.SemaphoreType.DMA((8,)),
        ],
        compiler_params=pltpu.CompilerParams(collective_id=0),
    )(t, W)


# device time: 42856 ns/iter; 1.0209x vs baseline; 1.0209x over previous
import jax
import jax.numpy as jnp
from jax import lax
from jax.experimental import pallas as pl
from jax.experimental.pallas import tpu as pltpu

N_DEV = 8

PIPES = (
    ((1, 2, 4), 0, 96),
    ((2, 4, 1), 96, 80),
    ((4, 1, 2), 176, 80),
)
RS_BASE = (0, 4, 6)
AG_BASE = (0, 1, 3)


def _span(masks):
    out = [0]
    for m in masks:
        out = out + [e ^ m for e in out]
    return out


def kernel(t, W):
    m, k = t.shape
    _, n = W.shape
    m_per = m // N_DEV

    def body(
        t_ref,
        w_hbm_ref,
        out_ref,
        acc_ref,
        w_f32_ref,
        w_bf_ref,
        recv_ref,
        rs_send_sems,
        rs_recv_sems,
        ag_send_sems,
        ag_recv_sems,
        w_dma_sem,
    ):
        i = lax.axis_index("i")
        l = i ^ ((i >> 1) & 1)

        def partner(mask):
            lp = l ^ mask
            return lp ^ ((lp >> 1) & 1)

        w_dma = pltpu.make_async_copy(w_hbm_ref, w_f32_ref, w_dma_sem)
        w_dma.start()

        barrier_sem = pltpu.get_barrier_semaphore()
        for mask in (1, 2, 4):
            pl.semaphore_signal(
                barrier_sem,
                inc=1,
                device_id=(partner(mask),),
                device_id_type=pl.DeviceIdType.MESH,
            )
        acc_ref[...] = t_ref[...].astype(jnp.bfloat16)
        pl.semaphore_wait(barrier_sem, 3)

        def rs_slot(p, s, e):
            return RS_BASE[s] + _span(PIPES[p][0][s + 1 :]).index(e)

        def rs_descriptor(p, s, e):
            order, roff, h = PIPES[p]
            mask = order[s]
            slot = rs_slot(p, s, e)
            c = l ^ mask ^ e
            return pltpu.make_async_remote_copy(
                src_ref=acc_ref.at[pl.ds(c * m_per + roff, h), :],
                dst_ref=recv_ref.at[slot, pl.ds(roff, h), :],
                send_sem=rs_send_sems.at[p, slot],
                recv_sem=rs_recv_sems.at[p, slot],
                device_id=(partner(mask),),
                device_id_type=pl.DeviceIdType.MESH,
            )

        def rs_recv(p, s, e):
            rs_descriptor(p, s, e).wait()
            roff, h = PIPES[p][1], PIPES[p][2]
            return recv_ref[rs_slot(p, s, e), pl.ds(roff, h), :].astype(
                jnp.float32
            )

        def band(c, p):
            roff, h = PIPES[p][1], PIPES[p][2]
            return pl.ds(c * m_per + roff, h)

        def acc_f32(c, p):
            return acc_ref[band(c, p), :].astype(jnp.float32)

        def ag_descriptor(p, s, e, recv):
            order, roff, h = PIPES[p]
            rev = order[::-1]
            mask = rev[s]
            slot = AG_BASE[s] + _span(rev[:s]).index(e)
            c = (l ^ mask ^ e) if recv else (l ^ e)
            rows = pl.ds(c * m_per + roff, h)
            return pltpu.make_async_remote_copy(
                src_ref=out_ref.at[rows, :],
                dst_ref=out_ref.at[rows, :],
                send_sem=ag_send_sems.at[p, slot],
                recv_sem=ag_recv_sems.at[p, slot],
                device_id=(partner(mask),),
                device_id_type=pl.DeviceIdType.MESH,
            )

        def ag_recv(p, s, e):
            ag_descriptor(p, s, e, recv=True).wait()

        for p in range(3):
            ms1, ms2 = PIPES[p][0][1], PIPES[p][0][2]
            for e in (ms1, ms1 ^ ms2, ms2, 0):
                rs_descriptor(p, 0, e).start()

        for p in range(3):
            ms1, ms2 = PIPES[p][0][1], PIPES[p][0][2]
            for e in (ms1, ms1 ^ ms2):
                acc_ref[band(l ^ e, p), :] = (
                    acc_f32(l ^ e, p) + rs_recv(p, 0, e)
                ).astype(jnp.bfloat16)
            for e in (ms2, 0):
                rs_descriptor(p, 1, e).start()

        for p in range(3):
            ms2 = PIPES[p][0][2]
            acc_ref[band(l ^ ms2, p), :] = (
                acc_f32(l ^ ms2, p) + rs_recv(p, 0, ms2) + rs_recv(p, 1, ms2)
            ).astype(jnp.bfloat16)
            rs_descriptor(p, 2, 0).start()

        w_dma.wait()
        w_bf_ref[...] = w_f32_ref[...].astype(jnp.bfloat16)

        for p in range(3):
            reduced = (
                acc_f32(l, p)
                + rs_recv(p, 0, 0)
                + rs_recv(p, 1, 0)
                + rs_recv(p, 2, 0)
            ).astype(jnp.bfloat16)
            res = jnp.dot(
                reduced, w_bf_ref[...], preferred_element_type=jnp.float32
            )
            out_ref[band(l, p), :] = res.astype(jnp.bfloat16)
            for s in (0, 1, 2):
                ag_descriptor(p, s, 0, recv=False).start()

        for p in range(3):
            rev = PIPES[p][0][::-1]
            ag_recv(p, 0, 0)
            ag_descriptor(p, 1, rev[0], recv=False).start()
            ag_descriptor(p, 2, rev[0], recv=False).start()
        for p in range(3):
            rev = PIPES[p][0][::-1]
            ag_recv(p, 1, 0)
            ag_descriptor(p, 2, rev[1], recv=False).start()
            ag_recv(p, 1, rev[0])
            ag_descriptor(p, 2, rev[1] ^ rev[0], recv=False).start()
        for p in range(3):
            rev = PIPES[p][0][::-1]
            for e in _span(rev[:2]):
                ag_recv(p, 2, e)

    return pl.pallas_call(
        body,
        out_shape=jax.ShapeDtypeStruct((m, n), jnp.bfloat16),
        in_specs=[
            pl.BlockSpec(memory_space=pltpu.VMEM),
            pl.BlockSpec(memory_space=pl.ANY),
        ],
        out_specs=pl.BlockSpec(memory_space=pltpu.VMEM),
        scratch_shapes=[
            pltpu.VMEM((m, k), jnp.bfloat16),
            pltpu.VMEM((k, n), jnp.float32),
            pltpu.VMEM((k, n), jnp.bfloat16),
            pltpu.VMEM((7, m_per, k), jnp.bfloat16),
            pltpu.SemaphoreType.DMA((3, 7)),
            pltpu.SemaphoreType.DMA((3, 7)),
            pltpu.SemaphoreType.DMA((3, 7)),
            pltpu.SemaphoreType.DMA((3, 7)),
            pltpu.SemaphoreType.DMA,
        ],
        compiler_params=pltpu.CompilerParams(collective_id=0),
    )(t, W)
